# Initial kernel scaffold; baseline (speedup 1.0000x reference)
import jax
import jax.numpy as jnp
from jax.experimental import pallas as pl


def kernel(logits, target):
    raise NotImplementedError("write your pallas kernel here")


# speedup vs baseline: 14.6791x; 14.6791x over previous
import functools

import jax
import jax.numpy as jnp
from jax.experimental import pallas as pl
from jax.experimental.pallas import tpu as pltpu

_SENT = 0x7F800000
_HI0 = 0x40000000


def _body(logits_ref, target_ref, eps_ref, out_ref, keys_ref, probs_ref):
    i = pl.program_id(0)

    l0 = logits_ref[0, 0]
    l1 = logits_ref[0, 1]
    m = jnp.maximum(l0, l1)
    e0 = jnp.exp(l0 - m)
    e1 = jnp.exp(l1 - m)
    p = e1 / (e0 + e1)
    t = target_ref[0, 0].astype(jnp.float32)
    tp = p * (t + eps_ref[0])
    fg = t == 1.0
    keys = jnp.where(fg, jax.lax.bitcast_convert_type(tp, jnp.int32),
                     jnp.int32(_SENT))
    keys_ref[...] = keys
    probs_ref[...] = p

    n_fg = jnp.sum(fg.astype(jnp.int32))
    k_num = jnp.maximum(jnp.int32(1), n_fg // 2)

    def step(_, lohi):
        lo, hi = lohi
        mid = (lo + hi) // 2
        cnt = jnp.sum((keys_ref[...] <= mid).astype(jnp.int32))
        ge = cnt >= k_num
        return jnp.where(ge, lo, mid + 1), jnp.where(ge, mid, hi)

    thr_key, _ = jax.lax.fori_loop(0, 31, step,
                                   (jnp.int32(0), jnp.int32(_HI0)))

    keys2 = keys_ref[...]
    p2 = probs_ref[...]
    fg2 = keys2 != jnp.int32(_SENT)
    kept = keys2 <= thr_key
    s_all = jnp.sum(p2)
    s_fg = jnp.sum(jnp.where(fg2, p2, 0.0))
    s_kept = jnp.sum(jnp.where(kept, p2, 0.0))
    c_kept = jnp.sum(kept.astype(jnp.int32)).astype(jnp.float32)

    inter = s_kept
    union = s_all - s_fg + s_kept + c_kept
    dice = jnp.where(union == 0.0, 1.0,
                     2.0 * inter / jnp.maximum(union, 1e-6))

    @pl.when(i == 0)
    def _():
        out_ref[...] = jnp.zeros_like(out_ref)

    out_ref[...] = out_ref[...] + dice

    @pl.when(i == pl.num_programs(0) - 1)
    def _():
        out_ref[...] = 1.0 - out_ref[...] / pl.num_programs(0)


def kernel(logits, target):
    b = logits.shape[0]
    h, w = logits.shape[2], logits.shape[3]
    eps_key = jax.random.key(42)
    epsilon = (jax.random.uniform(eps_key, (b, h * w), dtype=jnp.float32)
               * 1e-06).reshape(b, h, w)

    res = pl.pallas_call(
        _body,
        grid=(b,),
        in_specs=[
            pl.BlockSpec((1, 2, h, w), lambda i: (i, 0, 0, 0)),
            pl.BlockSpec((1, 1, h, w), lambda i: (i, 0, 0, 0)),
            pl.BlockSpec((1, h, w), lambda i: (i, 0, 0)),
        ],
        out_specs=pl.BlockSpec((1, 1), lambda i: (0, 0)),
        out_shape=jax.ShapeDtypeStruct((1, 1), jnp.float32),
        scratch_shapes=[
            pltpu.VMEM((h, w), jnp.int32),
            pltpu.VMEM((h, w), jnp.float32),
        ],
    )(logits, target, epsilon)
    return res[0, 0]
